# Initial kernel scaffold; baseline (speedup 1.0000x reference)
import jax
import jax.numpy as jnp
from jax.experimental import pallas as pl


def kernel(x, encoder, dec_w, dec_b):
    raise NotImplementedError("write your pallas kernel here")


# speedup vs baseline: 6.6184x; 6.6184x over previous
import jax
import jax.numpy as jnp
from jax.experimental import pallas as pl

_K = 32
_BN1 = 1024
_BH = 2048
_BN2 = 128
_BN3 = 256
_BH3 = 2048


def _acts_kernel(x_ref, enc_ref, out_ref):
    cos = jax.lax.dot_general(
        x_ref[...], enc_ref[...], (((1,), (1,)), ((), ())),
        preferred_element_type=jnp.float32,
    )
    out_ref[...] = jnp.clip(cos, -1.0, 1.0)


def _thresh_kernel(acts_ref, t_ref):
    m0 = jnp.max(acts_ref[...], axis=1)

    def body(_, m):
        masked = jnp.where(acts_ref[...] < m[:, None], acts_ref[...], -jnp.inf)
        return jnp.max(masked, axis=1)

    t = jax.lax.fori_loop(0, _K - 1, body, m0)
    t_ref[...] = jnp.broadcast_to(t[:, None], t_ref.shape)


def _decode_kernel(acts_ref, t_ref, dec_w_ref, dec_b_ref, recon_ref, sparse_ref):
    j = pl.program_id(1)
    t = t_ref[:, :1]
    cos = acts_ref[...]
    acts = 2.0 - jnp.sqrt(2.0 - 2.0 * cos)
    sparse = jnp.where(cos >= t, acts, 0.0)
    sparse_ref[...] = sparse
    partial = jax.lax.dot_general(
        sparse, dec_w_ref[...], (((1,), (1,)), ((), ())),
        preferred_element_type=jnp.float32,
    )

    @pl.when(j == 0)
    def _():
        recon_ref[...] = partial + dec_b_ref[...]

    @pl.when(j > 0)
    def _():
        recon_ref[...] += partial


def kernel(x, encoder, dec_w, dec_b):
    n, d = x.shape
    h = encoder.shape[0]

    w_norm = encoder / jnp.clip(jnp.linalg.norm(encoder, axis=1, keepdims=True), 1e-12)
    x_norm = x / jnp.clip(jnp.linalg.norm(x, axis=-1, keepdims=True), 1e-12)

    acts = pl.pallas_call(
        _acts_kernel,
        grid=(n // _BN1, h // _BH),
        in_specs=[
            pl.BlockSpec((_BN1, d), lambda i, j: (i, 0)),
            pl.BlockSpec((_BH, d), lambda i, j: (j, 0)),
        ],
        out_specs=pl.BlockSpec((_BN1, _BH), lambda i, j: (i, j)),
        out_shape=jax.ShapeDtypeStruct((n, h), jnp.float32),
    )(x_norm, w_norm)

    t = pl.pallas_call(
        _thresh_kernel,
        grid=(n // _BN2,),
        in_specs=[pl.BlockSpec((_BN2, h), lambda i: (i, 0))],
        out_specs=pl.BlockSpec((_BN2, 128), lambda i: (i, 0)),
        out_shape=jax.ShapeDtypeStruct((n, 128), jnp.float32),
    )(acts)

    recon, sparse = pl.pallas_call(
        _decode_kernel,
        grid=(n // _BN3, h // _BH3),
        in_specs=[
            pl.BlockSpec((_BN3, _BH3), lambda i, j: (i, j)),
            pl.BlockSpec((_BN3, 128), lambda i, j: (i, 0)),
            pl.BlockSpec((d, _BH3), lambda i, j: (0, j)),
            pl.BlockSpec((1, d), lambda i, j: (0, 0)),
        ],
        out_specs=[
            pl.BlockSpec((_BN3, d), lambda i, j: (i, 0)),
            pl.BlockSpec((_BN3, _BH3), lambda i, j: (i, j)),
        ],
        out_shape=[
            jax.ShapeDtypeStruct((n, d), jnp.float32),
            jax.ShapeDtypeStruct((n, h), jnp.float32),
        ],
    )(acts, t, dec_w, dec_b.reshape(1, d))
    return recon, sparse


# speedup vs baseline: 6.9925x; 1.0565x over previous
import jax
import jax.numpy as jnp
from jax.experimental import pallas as pl

_K = 32
_BN1 = 1024
_BH = 2048
_BN2 = 128
_BN3 = 256
_BH3 = 2048


def _acts_kernel(x_ref, enc_ref, out_ref):
    cos = jax.lax.dot_general(
        x_ref[...], enc_ref[...], (((1,), (1,)), ((), ())),
        preferred_element_type=jnp.float32,
    )
    out_ref[...] = jnp.clip(cos, -1.0, 1.0)


def _thresh_kernel(acts_ref, t_ref):
    m0 = jnp.max(acts_ref[...], axis=1)

    def body(_, m):
        masked = jnp.where(acts_ref[...] < m[:, None], acts_ref[...], -jnp.inf)
        return jnp.max(masked, axis=1)

    t = jax.lax.fori_loop(0, _K - 1, body, m0)
    t_ref[...] = jnp.broadcast_to(t[:, None], t_ref.shape)


def _decode_kernel(acts_ref, t_ref, dec_w_ref, dec_b_ref, recon_ref, sparse_ref):
    j = pl.program_id(1)
    t = t_ref[:, :1]
    cos = acts_ref[...]
    acts = 2.0 - jnp.sqrt(2.0 - 2.0 * cos)
    sparse = jnp.where(cos >= t, acts, 0.0)
    sparse_ref[...] = sparse
    partial = jax.lax.dot_general(
        sparse.astype(jnp.bfloat16), dec_w_ref[...], (((1,), (1,)), ((), ())),
        preferred_element_type=jnp.float32,
    )

    @pl.when(j == 0)
    def _():
        recon_ref[...] = partial + dec_b_ref[...]

    @pl.when(j > 0)
    def _():
        recon_ref[...] += partial


def kernel(x, encoder, dec_w, dec_b):
    n, d = x.shape
    h = encoder.shape[0]

    w_norm = encoder / jnp.clip(jnp.linalg.norm(encoder, axis=1, keepdims=True), 1e-12)
    x_norm = x / jnp.clip(jnp.linalg.norm(x, axis=-1, keepdims=True), 1e-12)
    x_norm = x_norm.astype(jnp.bfloat16)
    w_norm = w_norm.astype(jnp.bfloat16)
    dec_w_b = dec_w.astype(jnp.bfloat16)

    acts = pl.pallas_call(
        _acts_kernel,
        grid=(n // _BN1, h // _BH),
        in_specs=[
            pl.BlockSpec((_BN1, d), lambda i, j: (i, 0)),
            pl.BlockSpec((_BH, d), lambda i, j: (j, 0)),
        ],
        out_specs=pl.BlockSpec((_BN1, _BH), lambda i, j: (i, j)),
        out_shape=jax.ShapeDtypeStruct((n, h), jnp.float32),
    )(x_norm, w_norm)

    t = pl.pallas_call(
        _thresh_kernel,
        grid=(n // _BN2,),
        in_specs=[pl.BlockSpec((_BN2, h), lambda i: (i, 0))],
        out_specs=pl.BlockSpec((_BN2, 128), lambda i: (i, 0)),
        out_shape=jax.ShapeDtypeStruct((n, 128), jnp.float32),
    )(acts)

    recon, sparse = pl.pallas_call(
        _decode_kernel,
        grid=(n // _BN3, h // _BH3),
        in_specs=[
            pl.BlockSpec((_BN3, _BH3), lambda i, j: (i, j)),
            pl.BlockSpec((_BN3, 128), lambda i, j: (i, 0)),
            pl.BlockSpec((d, _BH3), lambda i, j: (0, j)),
            pl.BlockSpec((1, d), lambda i, j: (0, 0)),
        ],
        out_specs=[
            pl.BlockSpec((_BN3, d), lambda i, j: (i, 0)),
            pl.BlockSpec((_BN3, _BH3), lambda i, j: (i, j)),
        ],
        out_shape=[
            jax.ShapeDtypeStruct((n, d), jnp.float32),
            jax.ShapeDtypeStruct((n, h), jnp.float32),
        ],
    )(acts, t, dec_w_b, dec_b.reshape(1, d))
    return recon, sparse


# speedup vs baseline: 7.7236x; 1.1046x over previous
import jax
import jax.numpy as jnp
from jax.experimental import pallas as pl

_K = 32
_BN1 = 1024
_BH = 2048
_BN2 = 256
_BN3 = 512
_BH3 = 2048


def _acts_kernel(x_ref, enc_ref, out_ref):
    cos = jax.lax.dot_general(
        x_ref[...], enc_ref[...], (((1,), (1,)), ((), ())),
        preferred_element_type=jnp.float32,
    )
    out_ref[...] = jnp.clip(cos, -1.0, 1.0)


def _thresh_kernel(acts_ref, t_ref):
    m0 = jnp.max(acts_ref[...], axis=1)

    def body(_, m):
        masked = jnp.where(acts_ref[...] < m[:, None], acts_ref[...], -jnp.inf)
        return jnp.max(masked, axis=1)

    t = jax.lax.fori_loop(0, _K - 1, body, m0)
    t_ref[...] = jnp.broadcast_to(t[:, None], t_ref.shape)


def _decode_kernel(acts_ref, t_ref, dec_w_ref, dec_b_ref, recon_ref, sparse_ref):
    j = pl.program_id(1)
    t = t_ref[:, :1]
    cos = acts_ref[...]
    acts = 2.0 - jnp.sqrt(2.0 - 2.0 * cos)
    sparse = jnp.where(cos >= t, acts, 0.0)
    sparse_ref[...] = sparse
    partial = jax.lax.dot_general(
        sparse.astype(jnp.bfloat16), dec_w_ref[...], (((1,), (1,)), ((), ())),
        preferred_element_type=jnp.float32,
    )

    @pl.when(j == 0)
    def _():
        recon_ref[...] = partial + dec_b_ref[...]

    @pl.when(j > 0)
    def _():
        recon_ref[...] += partial


def kernel(x, encoder, dec_w, dec_b):
    n, d = x.shape
    h = encoder.shape[0]

    w_norm = encoder / jnp.clip(jnp.linalg.norm(encoder, axis=1, keepdims=True), 1e-12)
    x_norm = x / jnp.clip(jnp.linalg.norm(x, axis=-1, keepdims=True), 1e-12)
    x_norm = x_norm.astype(jnp.bfloat16)
    w_norm = w_norm.astype(jnp.bfloat16)
    dec_w_b = dec_w.astype(jnp.bfloat16)

    acts = pl.pallas_call(
        _acts_kernel,
        grid=(n // _BN1, h // _BH),
        in_specs=[
            pl.BlockSpec((_BN1, d), lambda i, j: (i, 0)),
            pl.BlockSpec((_BH, d), lambda i, j: (j, 0)),
        ],
        out_specs=pl.BlockSpec((_BN1, _BH), lambda i, j: (i, j)),
        out_shape=jax.ShapeDtypeStruct((n, h), jnp.float32),
    )(x_norm, w_norm)

    t = pl.pallas_call(
        _thresh_kernel,
        grid=(n // _BN2,),
        in_specs=[pl.BlockSpec((_BN2, h), lambda i: (i, 0))],
        out_specs=pl.BlockSpec((_BN2, 128), lambda i: (i, 0)),
        out_shape=jax.ShapeDtypeStruct((n, 128), jnp.float32),
    )(acts)

    recon, sparse = pl.pallas_call(
        _decode_kernel,
        grid=(n // _BN3, h // _BH3),
        in_specs=[
            pl.BlockSpec((_BN3, _BH3), lambda i, j: (i, j)),
            pl.BlockSpec((_BN3, 128), lambda i, j: (i, 0)),
            pl.BlockSpec((d, _BH3), lambda i, j: (0, j)),
            pl.BlockSpec((1, d), lambda i, j: (0, 0)),
        ],
        out_specs=[
            pl.BlockSpec((_BN3, d), lambda i, j: (i, 0)),
            pl.BlockSpec((_BN3, _BH3), lambda i, j: (i, j)),
        ],
        out_shape=[
            jax.ShapeDtypeStruct((n, d), jnp.float32),
            jax.ShapeDtypeStruct((n, h), jnp.float32),
        ],
    )(acts, t, dec_w_b, dec_b.reshape(1, d))
    return recon, sparse


# speedup vs baseline: 7.8902x; 1.0216x over previous
import jax
import jax.numpy as jnp
from jax.experimental import pallas as pl

_K = 32
_BN1 = 1024
_BH = 4096
_BN2 = 256
_BN3 = 512
_BH3 = 4096


def _acts_kernel(x_ref, enc_ref, out_ref):
    cos = jax.lax.dot_general(
        x_ref[...], enc_ref[...], (((1,), (1,)), ((), ())),
        preferred_element_type=jnp.float32,
    )
    out_ref[...] = jnp.clip(cos, -1.0, 1.0)


def _thresh_kernel(acts_ref, t_ref):
    m0 = jnp.max(acts_ref[...], axis=1)

    def body(_, m):
        masked = jnp.where(acts_ref[...] < m[:, None], acts_ref[...], -jnp.inf)
        return jnp.max(masked, axis=1)

    t = jax.lax.fori_loop(0, _K - 1, body, m0)
    t_ref[...] = jnp.broadcast_to(t[:, None], t_ref.shape)


def _decode_kernel(acts_ref, t_ref, dec_w_ref, dec_b_ref, recon_ref, sparse_ref):
    j = pl.program_id(1)
    t = t_ref[:, :1]
    cos = acts_ref[...]
    acts = 2.0 - jnp.sqrt(2.0 - 2.0 * cos)
    sparse = jnp.where(cos >= t, acts, 0.0)
    sparse_ref[...] = sparse
    partial = jax.lax.dot_general(
        sparse.astype(jnp.bfloat16), dec_w_ref[...], (((1,), (1,)), ((), ())),
        preferred_element_type=jnp.float32,
    )

    @pl.when(j == 0)
    def _():
        recon_ref[...] = partial + dec_b_ref[...]

    @pl.when(j > 0)
    def _():
        recon_ref[...] += partial


def kernel(x, encoder, dec_w, dec_b):
    n, d = x.shape
    h = encoder.shape[0]

    w_norm = encoder / jnp.clip(jnp.linalg.norm(encoder, axis=1, keepdims=True), 1e-12)
    x_norm = x / jnp.clip(jnp.linalg.norm(x, axis=-1, keepdims=True), 1e-12)
    x_norm = x_norm.astype(jnp.bfloat16)
    w_norm = w_norm.astype(jnp.bfloat16)
    dec_w_b = dec_w.astype(jnp.bfloat16)

    acts = pl.pallas_call(
        _acts_kernel,
        grid=(n // _BN1, h // _BH),
        in_specs=[
            pl.BlockSpec((_BN1, d), lambda i, j: (i, 0)),
            pl.BlockSpec((_BH, d), lambda i, j: (j, 0)),
        ],
        out_specs=pl.BlockSpec((_BN1, _BH), lambda i, j: (i, j)),
        out_shape=jax.ShapeDtypeStruct((n, h), jnp.float32),
    )(x_norm, w_norm)

    t = pl.pallas_call(
        _thresh_kernel,
        grid=(n // _BN2,),
        in_specs=[pl.BlockSpec((_BN2, h), lambda i: (i, 0))],
        out_specs=pl.BlockSpec((_BN2, 128), lambda i: (i, 0)),
        out_shape=jax.ShapeDtypeStruct((n, 128), jnp.float32),
    )(acts)

    recon, sparse = pl.pallas_call(
        _decode_kernel,
        grid=(n // _BN3, h // _BH3),
        in_specs=[
            pl.BlockSpec((_BN3, _BH3), lambda i, j: (i, j)),
            pl.BlockSpec((_BN3, 128), lambda i, j: (i, 0)),
            pl.BlockSpec((d, _BH3), lambda i, j: (0, j)),
            pl.BlockSpec((1, d), lambda i, j: (0, 0)),
        ],
        out_specs=[
            pl.BlockSpec((_BN3, d), lambda i, j: (i, 0)),
            pl.BlockSpec((_BN3, _BH3), lambda i, j: (i, j)),
        ],
        out_shape=[
            jax.ShapeDtypeStruct((n, d), jnp.float32),
            jax.ShapeDtypeStruct((n, h), jnp.float32),
        ],
    )(acts, t, dec_w_b, dec_b.reshape(1, d))
    return recon, sparse
